# baseline (speedup 1.0000x reference)
import functools

import jax
import jax.numpy as jnp
from jax import lax
from jax.experimental import pallas as pl
from jax.experimental.pallas import tpu as pltpu
from jax.experimental.pallas import tpu_sc as plsc

_ROWS = 4096
_COLS = 4096
_NC = 2
_NS = 16
_NW = _NC * _NS
_L = 16

_TILE_ROWS = _ROWS // _NW
_CH = 4
_N_CH = _TILE_ROWS // _CH
_VECS_PER_ROW = _COLS // (4 * _L)


def _prune_body(x_hbm, out_hbm, bin0, bin1, bout0, bout1, si0, si1, so0, so1):
    wid = lax.axis_index("s") * _NC + lax.axis_index("c")
    row0 = wid * _TILE_ROWS
    iota4 = lax.iota(jnp.int32, _L) * 4
    one = jnp.float32(1.0)
    zero = jnp.float32(0.0)
    bins = (bin0, bin1)
    bouts = (bout0, bout1)
    sis = (si0, si1)
    sos = (so0, so1)

    def in_copy(ch, b):
        return pltpu.make_async_copy(
            x_hbm.at[pl.ds(row0 + ch * _CH, _CH)], bins[b], sis[b])

    def out_copy(ch, b):
        return pltpu.make_async_copy(
            bouts[b], out_hbm.at[pl.ds(row0 + ch * _CH, _CH)], sos[b])

    def compute(bin_, bout):
        def row_body(r, carry):
            rv = jnp.full((_L,), r, jnp.int32)

            def body(j, c):
                cols = iota4 + lax.shift_left(j, 6)
                s0 = plsc.load_gather(bin_, [rv, cols])
                s1 = plsc.load_gather(bin_, [rv, cols + 1])
                s2 = plsc.load_gather(bin_, [rv, cols + 2])
                s3 = plsc.load_gather(bin_, [rv, cols + 3])
                a0 = s0 * s0
                a1 = s1 * s1
                a2 = s2 * s2
                a3 = s3 * s3
                c01 = a0 >= a1
                c02 = a0 >= a2
                c03 = a0 >= a3
                c12 = a1 >= a2
                c13 = a1 >= a3
                c23 = a2 >= a3
                keep0 = (c01 & (c02 | c03)) | (c02 & c03)
                w10 = ~c01
                keep1 = (w10 & (c12 | c13)) | (c12 & c13)
                w20 = ~c02
                w21 = ~c12
                keep2 = (w20 & (w21 | c23)) | (w21 & c23)
                w30 = ~c03
                w31 = ~c13
                w32 = ~c23
                keep3 = (w30 & (w31 | w32)) | (w31 & w32)
                plsc.store_scatter(bout, [rv, cols],
                                   jnp.where(keep0, s0, zero))
                plsc.store_scatter(bout, [rv, cols + 1],
                                   jnp.where(keep1, s1, zero))
                plsc.store_scatter(bout, [rv, cols + 2],
                                   jnp.where(keep2, s2, zero))
                plsc.store_scatter(bout, [rv, cols + 3],
                                   jnp.where(keep3, s3, zero))
                return c

            lax.fori_loop(0, _VECS_PER_ROW, body, 0, unroll=True)
            return carry

        lax.fori_loop(0, _CH, row_body, 0)

    in_copy(0, 0).start()

    def ring(g, carry):
        for b in range(2):
            ch = g * 2 + b

            @pl.when(ch + 1 < _N_CH)
            def _():
                in_copy(ch + 1, 1 - b).start()

            in_copy(ch, b).wait()

            @pl.when(ch >= 2)
            def _():
                out_copy(ch - 2, b).wait()

            compute(bins[b], bouts[b])
            out_copy(ch, b).start()
        return carry

    lax.fori_loop(0, _N_CH // 2, ring, 0)
    out_copy(_N_CH - 2, 0).wait()
    out_copy(_N_CH - 1, 1).wait()


_prune = functools.partial(
    pl.kernel,
    out_type=jax.ShapeDtypeStruct((_ROWS, _COLS), jnp.float32),
    mesh=plsc.VectorSubcoreMesh(core_axis_name="c", subcore_axis_name="s"),
    scratch_types=[
        pltpu.VMEM((_CH, _COLS), jnp.float32),
        pltpu.VMEM((_CH, _COLS), jnp.float32),
        pltpu.VMEM((_CH, _COLS), jnp.float32),
        pltpu.VMEM((_CH, _COLS), jnp.float32),
        pltpu.SemaphoreType.DMA,
        pltpu.SemaphoreType.DMA,
        pltpu.SemaphoreType.DMA,
        pltpu.SemaphoreType.DMA,
    ],
    compiler_params=pltpu.CompilerParams(needs_layout_passes=False),
)(_prune_body)


def kernel(x, bias):
    return _prune(x), bias


# speedup vs baseline: 2.6064x; 2.6064x over previous
import functools

import jax
import jax.numpy as jnp
from jax import lax
from jax.experimental import pallas as pl
from jax.experimental.pallas import tpu as pltpu
from jax.experimental.pallas import tpu_sc as plsc

_ROWS = 4096
_COLS = 4096
_NC = 2
_NS = 16
_NW = _NC * _NS
_L = 16

_TILE_ROWS = _ROWS // _NW
_CH = 4
_N_CH = _TILE_ROWS // _CH
_CHW = _CH * _COLS
_SPANS = _CHW // (4 * _L)


def _prune_body(x_hbm, out_hbm, bin0, bin1, bout0, bout1, si0, si1, so0, so1):
    wid = lax.axis_index("s") * _NC + lax.axis_index("c")
    row0 = wid * _TILE_ROWS
    iota4 = lax.iota(jnp.int32, _L) * 4
    one = jnp.float32(1.0)
    zero = jnp.float32(0.0)
    bins = (bin0, bin1)
    bouts = (bout0, bout1)
    sis = (si0, si1)
    sos = (so0, so1)

    def in_copies(ch, b):
        return [
            pltpu.make_async_copy(
                x_hbm.at[row0 + ch * _CH + r],
                bins[b].at[pl.ds(r * _COLS, _COLS)], sis[b])
            for r in range(_CH)
        ]

    def out_copies(ch, b):
        return [
            pltpu.make_async_copy(
                bouts[b].at[pl.ds(r * _COLS, _COLS)],
                out_hbm.at[row0 + ch * _CH + r], sos[b])
            for r in range(_CH)
        ]

    def compute(bin_, bout):
        def body(j, c):
            f0 = iota4 + lax.shift_left(j, 6)
            f1 = f0 + 1
            f2 = f0 + 2
            f3 = f0 + 3
            s0 = plsc.load_gather(bin_, [f0])
            s1 = plsc.load_gather(bin_, [f1])
            s2 = plsc.load_gather(bin_, [f2])
            s3 = plsc.load_gather(bin_, [f3])
            a0 = s0 * s0
            a1 = s1 * s1
            a2 = s2 * s2
            a3 = s3 * s3
            n01 = jnp.where(a0 >= a1, one, zero)
            n02 = jnp.where(a0 >= a2, one, zero)
            n03 = jnp.where(a0 >= a3, one, zero)
            n12 = jnp.where(a1 >= a2, one, zero)
            n13 = jnp.where(a1 >= a3, one, zero)
            n23 = jnp.where(a2 >= a3, one, zero)
            keep0 = (n01 + n02 + n03) >= 2.0
            keep1 = (n12 + n13 - n01) >= 1.0
            keep2 = (n23 - n02 - n12) >= 0.0
            keep3 = (n03 + n13 + n23) <= 1.0
            plsc.store_scatter(bout, [f0], jnp.where(keep0, s0, zero))
            plsc.store_scatter(bout, [f1], jnp.where(keep1, s1, zero))
            plsc.store_scatter(bout, [f2], jnp.where(keep2, s2, zero))
            plsc.store_scatter(bout, [f3], jnp.where(keep3, s3, zero))
            return c

        lax.fori_loop(0, _SPANS, body, 0, unroll=16)

    for cp in in_copies(0, 0):
        cp.start()

    def ring(g, carry):
        for b in range(2):
            ch = g * 2 + b

            @pl.when(ch + 1 < _N_CH)
            def _():
                for cp in in_copies(ch + 1, 1 - b):
                    cp.start()

            for cp in in_copies(ch, b):
                cp.wait()

            @pl.when(ch >= 2)
            def _():
                for cp in out_copies(ch - 2, b):
                    cp.wait()

            compute(bins[b], bouts[b])
            for cp in out_copies(ch, b):
                cp.start()
        return carry

    lax.fori_loop(0, _N_CH // 2, ring, 0)
    for cp in out_copies(_N_CH - 2, 0):
        cp.wait()
    for cp in out_copies(_N_CH - 1, 1):
        cp.wait()


_prune = functools.partial(
    pl.kernel,
    out_type=jax.ShapeDtypeStruct((_ROWS, _COLS), jnp.float32),
    mesh=plsc.VectorSubcoreMesh(core_axis_name="c", subcore_axis_name="s"),
    scratch_types=[
        pltpu.VMEM((_CHW,), jnp.float32),
        pltpu.VMEM((_CHW,), jnp.float32),
        pltpu.VMEM((_CHW,), jnp.float32),
        pltpu.VMEM((_CHW,), jnp.float32),
        pltpu.SemaphoreType.DMA,
        pltpu.SemaphoreType.DMA,
        pltpu.SemaphoreType.DMA,
        pltpu.SemaphoreType.DMA,
    ],
    compiler_params=pltpu.CompilerParams(needs_layout_passes=False),
)(_prune_body)


def kernel(x, bias):
    return _prune(x), bias
